# Initial kernel scaffold; baseline (speedup 1.0000x reference)
import jax
import jax.numpy as jnp
from jax.experimental import pallas as pl


def kernel(node_features, edge_features, W1, Wr1, W2, Wr2, conv_w, fc_w, fc_b, fc1_w, fc1_b, edge_index):
    raise NotImplementedError("write your pallas kernel here")


# speedup vs baseline: 3.0684x; 3.0684x over previous
import functools

import jax
import jax.numpy as jnp
import numpy as np
from jax import lax
from jax.experimental import pallas as pl
from jax.experimental.pallas import tpu as pltpu
from jax.experimental.pallas import tpu_sc as plsc

N_NODES = 10000
N_EDGES = 320000
D = 128
N_CLASSES = 512
FLAT = 1600
KER = 7

N_PAD = 10240
ROWS_PER_TILE = N_PAD // 16
CHUNK = 128
EDGES_PER_TILE = N_EDGES // 16
FULL_CHUNKS = EDGES_PER_TILE // CHUNK
TAIL = EDGES_PER_TILE - FULL_CHUNKS * CHUNK
EDGES_PER_TILE_H = N_EDGES // 32
FULL_CHUNKS_H = EDGES_PER_TILE_H // CHUNK
TAIL_H = EDGES_PER_TILE_H - FULL_CHUNKS_H * CHUNK

_mesh = plsc.VectorSubcoreMesh(core_axis_name="c", subcore_axis_name="s")


def _zero_acc(zeros_hbm, zbuf_v, acc_sh, sid):
    pltpu.sync_copy(zeros_hbm, zbuf_v)
    for k in range(ROWS_PER_TILE // 128):
        pltpu.sync_copy(zbuf_v, acc_sh.at[pl.ds(sid * ROWS_PER_TILE + k * 128, 128)])


def _copy_out(acc_sh, zbuf_v, out_hbm, cid, sid):
    for k in range(ROWS_PER_TILE // 128):
        row = sid * ROWS_PER_TILE + k * 128
        pltpu.sync_copy(acc_sh.at[pl.ds(row, 128)], zbuf_v)
        pltpu.sync_copy(zbuf_v, out_hbm.at[cid, pl.ds(row, 128)])


def _sc_dual_body(x_hbm, vals_hbm, src_hbm, dst_hbm, zeros_hbm, out_hbm,
                  src_v, dst_v, srct_v, dstt_v, vals_v, zbuf_v, acc_sh, sem):
    cid = lax.axis_index("c")
    sid = lax.axis_index("s")
    _zero_acc(zeros_hbm, zbuf_v, acc_sh, sid)
    plsc.subcore_barrier()
    base = sid * EDGES_PER_TILE

    @pl.when(cid == 0)
    def _():
        def body(j, carry):
            off = base + j * CHUNK
            pltpu.sync_copy(src_hbm.at[pl.ds(off, CHUNK)], src_v)
            pltpu.async_copy(x_hbm.at[src_v], vals_v, sem).wait()
            pltpu.sync_copy(dst_hbm.at[pl.ds(off, CHUNK)], dst_v)
            pltpu.sync_copy(vals_v, acc_sh.at[dst_v], add=True)
            return carry
        lax.fori_loop(0, FULL_CHUNKS, body, 0)
        off = base + FULL_CHUNKS * CHUNK
        pltpu.sync_copy(src_hbm.at[pl.ds(off, TAIL)], srct_v)
        pltpu.async_copy(x_hbm.at[srct_v], vals_v.at[pl.ds(0, TAIL)], sem).wait()
        pltpu.sync_copy(dst_hbm.at[pl.ds(off, TAIL)], dstt_v)
        pltpu.sync_copy(vals_v.at[pl.ds(0, TAIL)], acc_sh.at[dstt_v], add=True)

    @pl.when(cid == 1)
    def _():
        def body(j, carry):
            off = base + j * CHUNK
            pltpu.sync_copy(vals_hbm.at[pl.ds(off, CHUNK)], vals_v)
            pltpu.sync_copy(dst_hbm.at[pl.ds(off, CHUNK)], dst_v)
            pltpu.sync_copy(vals_v, acc_sh.at[dst_v], add=True)
            return carry
        lax.fori_loop(0, FULL_CHUNKS, body, 0)
        off = base + FULL_CHUNKS * CHUNK
        pltpu.sync_copy(vals_hbm.at[pl.ds(off, TAIL)], vals_v.at[pl.ds(0, TAIL)])
        pltpu.sync_copy(dst_hbm.at[pl.ds(off, TAIL)], dstt_v)
        pltpu.sync_copy(vals_v.at[pl.ds(0, TAIL)], acc_sh.at[dstt_v], add=True)

    plsc.subcore_barrier()
    _copy_out(acc_sh, zbuf_v, out_hbm, cid, sid)


_sc_dual = pl.kernel(
    _sc_dual_body,
    out_type=jax.ShapeDtypeStruct((2, N_PAD, D), jnp.float32),
    mesh=_mesh,
    scratch_types=[
        pltpu.VMEM((CHUNK,), jnp.int32),
        pltpu.VMEM((CHUNK,), jnp.int32),
        pltpu.VMEM((TAIL,), jnp.int32),
        pltpu.VMEM((TAIL,), jnp.int32),
        pltpu.VMEM((CHUNK, D), jnp.float32),
        pltpu.VMEM((128, D), jnp.float32),
        pltpu.VMEM_SHARED((N_PAD, D), jnp.float32),
        pltpu.SemaphoreType.DMA,
    ],
)


def _sc_linear_body(vals_hbm, dst_hbm, zeros_hbm, out_hbm,
                    dst_v, dstt_v, vals_v, zbuf_v, acc_sh):
    cid = lax.axis_index("c")
    sid = lax.axis_index("s")
    _zero_acc(zeros_hbm, zbuf_v, acc_sh, sid)
    plsc.subcore_barrier()
    base = cid * (N_EDGES // 2) + sid * EDGES_PER_TILE_H

    def body(j, carry):
        off = base + j * CHUNK
        pltpu.sync_copy(vals_hbm.at[pl.ds(off, CHUNK)], vals_v)
        pltpu.sync_copy(dst_hbm.at[pl.ds(off, CHUNK)], dst_v)
        pltpu.sync_copy(vals_v, acc_sh.at[dst_v], add=True)
        return carry
    lax.fori_loop(0, FULL_CHUNKS_H, body, 0)
    off = base + FULL_CHUNKS_H * CHUNK
    pltpu.sync_copy(vals_hbm.at[pl.ds(off, TAIL_H)], vals_v.at[pl.ds(0, TAIL_H)])
    pltpu.sync_copy(dst_hbm.at[pl.ds(off, TAIL_H)], dstt_v)
    pltpu.sync_copy(vals_v.at[pl.ds(0, TAIL_H)], acc_sh.at[dstt_v], add=True)

    plsc.subcore_barrier()
    _copy_out(acc_sh, zbuf_v, out_hbm, cid, sid)


_sc_linear = pl.kernel(
    _sc_linear_body,
    out_type=jax.ShapeDtypeStruct((2, N_PAD, D), jnp.float32),
    mesh=_mesh,
    scratch_types=[
        pltpu.VMEM((CHUNK,), jnp.int32),
        pltpu.VMEM((TAIL_H,), jnp.int32),
        pltpu.VMEM((CHUNK, D), jnp.float32),
        pltpu.VMEM((128, D), jnp.float32),
        pltpu.VMEM_SHARED((N_PAD, D), jnp.float32),
    ],
)


_R23_BLK = 2560


def _tc_r23_body(r1_ref, wr1_ref, wr2_ref, r2_ref, r3_ref):
    r2 = jnp.tanh(jnp.dot(r1_ref[...], wr1_ref[...], preferred_element_type=jnp.float32))
    r3 = jnp.tanh(jnp.dot(r2, wr2_ref[...], preferred_element_type=jnp.float32))
    r2_ref[...] = r2
    r3_ref[...] = r3


def _tc_r23(r1, wr1, wr2):
    return pl.pallas_call(
        _tc_r23_body,
        grid=(N_EDGES // _R23_BLK,),
        in_specs=[
            pl.BlockSpec((_R23_BLK, D), lambda i: (i, 0)),
            pl.BlockSpec((D, D), lambda i: (0, 0)),
            pl.BlockSpec((D, D), lambda i: (0, 0)),
        ],
        out_specs=[
            pl.BlockSpec((_R23_BLK, D), lambda i: (i, 0)),
            pl.BlockSpec((_R23_BLK, D), lambda i: (i, 0)),
        ],
        out_shape=[
            jax.ShapeDtypeStruct((N_EDGES, D), jnp.float32),
            jax.ShapeDtypeStruct((N_EDGES, D), jnp.float32),
        ],
    )(r1, wr1, wr2)


_X_BLK = 2048


def _tc_xupd_body(sr_ref, w_ref, x_ref):
    d = sr_ref[0] - sr_ref[1]
    x_ref[...] = jnp.tanh(jnp.dot(d, w_ref[...], preferred_element_type=jnp.float32))


def _tc_xupd(sr, w):
    return pl.pallas_call(
        _tc_xupd_body,
        grid=(N_PAD // _X_BLK,),
        in_specs=[
            pl.BlockSpec((2, _X_BLK, D), lambda i: (0, i, 0)),
            pl.BlockSpec((D, D), lambda i: (0, 0)),
        ],
        out_specs=pl.BlockSpec((_X_BLK, D), lambda i: (i, 0)),
        out_shape=jax.ShapeDtypeStruct((N_PAD, D), jnp.float32),
    )(sr, w)


_H_BLK = 2000


def _tc_head_body(sr2_ref, r3p_ref, w2_ref, m2_ref, fcw_ref, fcb_ref,
                  fc1w_ref, fc1b_ref, out_ref):
    d = sr2_ref[0] - sr2_ref[1]
    x2 = jnp.tanh(jnp.dot(d, w2_ref[...], preferred_element_type=jnp.float32))
    rn = r3p_ref[0] + r3p_ref[1]
    h = jnp.concatenate([x2, rn], axis=1)
    h = jnp.maximum(jnp.dot(h, m2_ref[...], preferred_element_type=jnp.float32), 0.0)
    h = lax.dot_general(h, fcw_ref[...], (((1,), (1,)), ((), ())),
                        preferred_element_type=jnp.float32) + fcb_ref[...]
    h = jnp.maximum(h, 0.0)
    out_ref[...] = lax.dot_general(h, fc1w_ref[...], (((1,), (1,)), ((), ())),
                                   preferred_element_type=jnp.float32) + fc1b_ref[...]


def _tc_head(sr2, r3p, w2, m2, fc_w, fc_b, fc1_w, fc1_b):
    return pl.pallas_call(
        _tc_head_body,
        grid=(N_NODES // _H_BLK,),
        in_specs=[
            pl.BlockSpec((2, _H_BLK, D), lambda i: (0, i, 0)),
            pl.BlockSpec((2, _H_BLK, D), lambda i: (0, i, 0)),
            pl.BlockSpec((D, D), lambda i: (0, 0)),
            pl.BlockSpec((2 * D, FLAT), lambda i: (0, 0)),
            pl.BlockSpec((D, FLAT), lambda i: (0, 0)),
            pl.BlockSpec((1, D), lambda i: (0, 0)),
            pl.BlockSpec((N_CLASSES, D), lambda i: (0, 0)),
            pl.BlockSpec((1, N_CLASSES), lambda i: (0, 0)),
        ],
        out_specs=pl.BlockSpec((_H_BLK, N_CLASSES), lambda i: (i, 0)),
        out_shape=jax.ShapeDtypeStruct((N_NODES, N_CLASSES), jnp.float32),
    )(sr2, r3p, w2, m2, fc_w, fc_b, fc1_w, fc1_b)


def _conv_map():
    rows, cols, widx = [], [], []
    for f in range(16):
        for i in range(10):
            for j in range(10):
                for a in range(KER):
                    for b in range(KER):
                        l = 16 * (i + a) + (j + b)
                        rows.append((l % 2) * D + l // 2)
                        cols.append(f * 100 + i * 10 + j)
                        widx.append(f * KER * KER + a * KER + b)
    return (np.asarray(rows, np.int32), np.asarray(cols, np.int32),
            np.asarray(widx, np.int32))


_CONV_ROWS, _CONV_COLS, _CONV_WIDX = _conv_map()


def kernel(node_features, edge_features, W1, Wr1, W2, Wr2, conv_w, fc_w, fc_b,
           fc1_w, fc1_b, edge_index):
    src = edge_index[0].astype(jnp.int32)
    dst = edge_index[1].astype(jnp.int32)
    zeros128 = jnp.zeros((128, D), jnp.float32)

    m2 = jnp.zeros((2 * D, FLAT), jnp.float32).at[_CONV_ROWS, _CONV_COLS].add(
        conv_w.reshape(-1)[_CONV_WIDX])

    r2, r3 = _tc_r23(edge_features, Wr1, Wr2)
    sr1 = _sc_dual(node_features, edge_features, src, dst, zeros128)
    x1 = _tc_xupd(sr1, W1)
    sr2 = _sc_dual(x1, r2, src, dst, zeros128)
    r3p = _sc_linear(r3, dst, zeros128)
    return _tc_head(sr2, r3p, W2, m2, fc_w, fc_b.reshape(1, D),
                    fc1_w, fc1_b.reshape(1, N_CLASSES))


# speedup vs baseline: 5.0802x; 1.6556x over previous
import jax
import jax.numpy as jnp
import numpy as np
from jax import lax
from jax.experimental import pallas as pl
from jax.experimental.pallas import tpu as pltpu
from jax.experimental.pallas import tpu_sc as plsc

N_NODES = 10000
N_EDGES = 320000
D = 128
N_CLASSES = 512
FLAT = 1600
KER = 7

N_PAD = 10240
ROWS_PER_TILE = N_PAD // 16
CHUNK = 128

EDGES_PER_TILE = N_EDGES // 16
FULL_CHUNKS = EDGES_PER_TILE // CHUNK
TAIL = EDGES_PER_TILE - FULL_CHUNKS * CHUNK
NBUF = 2
GROUPS = FULL_CHUNKS // NBUF

EDGES_PER_TILE_H = N_EDGES // 32
FULL_CHUNKS_H = EDGES_PER_TILE_H // CHUNK
TAIL_H = EDGES_PER_TILE_H - FULL_CHUNKS_H * CHUNK
NBUF_H = 2
GROUPS_H = FULL_CHUNKS_H // NBUF_H

_mesh = plsc.VectorSubcoreMesh(core_axis_name="c", subcore_axis_name="s")


def _zero_acc(zeros_hbm, acc_sh, sid, osem):
    for k in range(ROWS_PER_TILE // 128):
        pltpu.async_copy(zeros_hbm, acc_sh.at[pl.ds(sid * ROWS_PER_TILE + k * 128, 128)], osem)
    for k in range(ROWS_PER_TILE // 128):
        pltpu.make_async_copy(zeros_hbm, acc_sh.at[pl.ds(sid * ROWS_PER_TILE, 128)], osem).wait()


def _copy_out(acc_sh, out_hbm, cid, sid, osem):
    for k in range(ROWS_PER_TILE // 128):
        row = sid * ROWS_PER_TILE + k * 128
        pltpu.async_copy(acc_sh.at[pl.ds(row, 128)], out_hbm.at[cid, pl.ds(row, 128)], osem)
    for k in range(ROWS_PER_TILE // 128):
        pltpu.make_async_copy(acc_sh.at[pl.ds(0, 128)], out_hbm.at[cid, pl.ds(0, 128)], osem).wait()


def _gather_sweep(x_hbm, src_hbm, dst_hbm, acc_sh, src_v, dst_v, vals_v,
                  isem, gsem, ssem, base, groups, nbuf):
    def grp(g, carry):
        @pl.when(g > 0)
        def _():
            for b in range(nbuf):
                pltpu.make_async_copy(vals_v.at[b], acc_sh.at[dst_v.at[b]], ssem.at[b]).wait()
        for b in range(nbuf):
            off = base + (g * nbuf + b) * CHUNK
            pltpu.async_copy(src_hbm.at[pl.ds(off, CHUNK)], src_v.at[b], isem.at[b])
            pltpu.async_copy(dst_hbm.at[pl.ds(off, CHUNK)], dst_v.at[b], isem.at[b])
        for b in range(nbuf):
            pltpu.make_async_copy(src_hbm.at[pl.ds(0, CHUNK)], src_v.at[b], isem.at[b]).wait()
            pltpu.async_copy(x_hbm.at[src_v.at[b]], vals_v.at[b], gsem.at[b])
        for b in range(nbuf):
            pltpu.make_async_copy(x_hbm.at[src_v.at[b]], vals_v.at[b], gsem.at[b]).wait()
            pltpu.make_async_copy(dst_hbm.at[pl.ds(0, CHUNK)], dst_v.at[b], isem.at[b]).wait()
            pltpu.async_copy(vals_v.at[b], acc_sh.at[dst_v.at[b]], ssem.at[b], add=True)
        return carry
    lax.fori_loop(0, groups, grp, 0)
    for b in range(nbuf):
        pltpu.make_async_copy(vals_v.at[b], acc_sh.at[dst_v.at[b]], ssem.at[b]).wait()


def _linear_sweep(vals_hbm, dst_hbm, acc_sh, dst_v, vals_v,
                  isem, gsem, ssem, base, groups, nbuf):
    def grp(g, carry):
        @pl.when(g > 0)
        def _():
            for b in range(nbuf):
                pltpu.make_async_copy(vals_v.at[b], acc_sh.at[dst_v.at[b]], ssem.at[b]).wait()
        for b in range(nbuf):
            off = base + (g * nbuf + b) * CHUNK
            pltpu.async_copy(dst_hbm.at[pl.ds(off, CHUNK)], dst_v.at[b], isem.at[b])
            pltpu.async_copy(vals_hbm.at[pl.ds(off, CHUNK)], vals_v.at[b], gsem.at[b])
        for b in range(nbuf):
            pltpu.make_async_copy(vals_hbm.at[pl.ds(0, CHUNK)], vals_v.at[b], gsem.at[b]).wait()
            pltpu.make_async_copy(dst_hbm.at[pl.ds(0, CHUNK)], dst_v.at[b], isem.at[b]).wait()
            pltpu.async_copy(vals_v.at[b], acc_sh.at[dst_v.at[b]], ssem.at[b], add=True)
        return carry
    lax.fori_loop(0, groups, grp, 0)
    for b in range(nbuf):
        pltpu.make_async_copy(vals_v.at[b], acc_sh.at[dst_v.at[b]], ssem.at[b]).wait()


def _gather_tail(x_hbm, src_hbm, dst_hbm, acc_sh, srct_v, dstt_v, vals_v, gsem, off, tail):
    pltpu.sync_copy(src_hbm.at[pl.ds(off, tail)], srct_v)
    pltpu.async_copy(x_hbm.at[srct_v], vals_v.at[0, pl.ds(0, tail)], gsem.at[0]).wait()
    pltpu.sync_copy(dst_hbm.at[pl.ds(off, tail)], dstt_v)
    pltpu.sync_copy(vals_v.at[0, pl.ds(0, tail)], acc_sh.at[dstt_v], add=True)


def _linear_tail(vals_hbm, dst_hbm, acc_sh, dstt_v, vals_v, off, tail):
    pltpu.sync_copy(vals_hbm.at[pl.ds(off, tail)], vals_v.at[0, pl.ds(0, tail)])
    pltpu.sync_copy(dst_hbm.at[pl.ds(off, tail)], dstt_v)
    pltpu.sync_copy(vals_v.at[0, pl.ds(0, tail)], acc_sh.at[dstt_v], add=True)


def _sc_dual_body(x_hbm, vals_hbm, src_hbm, dst_hbm, zeros_hbm, out_hbm,
                  src_v, dst_v, srct_v, dstt_v, vals_v, acc_sh,
                  isem, gsem, ssem, osem):
    cid = lax.axis_index("c")
    sid = lax.axis_index("s")
    _zero_acc(zeros_hbm, acc_sh, sid, osem)
    plsc.subcore_barrier()
    base = sid * EDGES_PER_TILE
    toff = base + FULL_CHUNKS * CHUNK

    @pl.when(cid == 0)
    def _():
        _gather_sweep(x_hbm, src_hbm, dst_hbm, acc_sh, src_v, dst_v, vals_v,
                      isem, gsem, ssem, base, GROUPS, NBUF)
        _gather_tail(x_hbm, src_hbm, dst_hbm, acc_sh, srct_v, dstt_v, vals_v, gsem, toff, TAIL)

    @pl.when(cid == 1)
    def _():
        _linear_sweep(vals_hbm, dst_hbm, acc_sh, dst_v, vals_v,
                      isem, gsem, ssem, base, GROUPS, NBUF)
        _linear_tail(vals_hbm, dst_hbm, acc_sh, dstt_v, vals_v, toff, TAIL)

    plsc.subcore_barrier()
    _copy_out(acc_sh, out_hbm, cid, sid, osem)


_sc_dual = pl.kernel(
    _sc_dual_body,
    out_type=jax.ShapeDtypeStruct((2, N_PAD, D), jnp.float32),
    mesh=_mesh,
    scratch_types=[
        pltpu.VMEM((NBUF, CHUNK), jnp.int32),
        pltpu.VMEM((NBUF, CHUNK), jnp.int32),
        pltpu.VMEM((TAIL,), jnp.int32),
        pltpu.VMEM((TAIL,), jnp.int32),
        pltpu.VMEM((NBUF, CHUNK, D), jnp.float32),
        pltpu.VMEM_SHARED((N_PAD, D), jnp.float32),
        pltpu.SemaphoreType.DMA((NBUF,)),
        pltpu.SemaphoreType.DMA((NBUF,)),
        pltpu.SemaphoreType.DMA((NBUF,)),
        pltpu.SemaphoreType.DMA,
    ],
)


def _sc_linear_body(vals_hbm, dst_hbm, zeros_hbm, out_hbm,
                    dst_v, dstt_v, vals_v, acc_sh, isem, gsem, ssem, osem):
    cid = lax.axis_index("c")
    sid = lax.axis_index("s")
    _zero_acc(zeros_hbm, acc_sh, sid, osem)
    plsc.subcore_barrier()
    base = cid * (N_EDGES // 2) + sid * EDGES_PER_TILE_H
    _linear_sweep(vals_hbm, dst_hbm, acc_sh, dst_v, vals_v,
                  isem, gsem, ssem, base, GROUPS_H, NBUF_H)
    _linear_tail(vals_hbm, dst_hbm, acc_sh, dstt_v, vals_v,
                 base + FULL_CHUNKS_H * CHUNK, TAIL_H)
    plsc.subcore_barrier()
    _copy_out(acc_sh, out_hbm, cid, sid, osem)


_sc_linear = pl.kernel(
    _sc_linear_body,
    out_type=jax.ShapeDtypeStruct((2, N_PAD, D), jnp.float32),
    mesh=_mesh,
    scratch_types=[
        pltpu.VMEM((NBUF_H, CHUNK), jnp.int32),
        pltpu.VMEM((TAIL_H,), jnp.int32),
        pltpu.VMEM((NBUF_H, CHUNK, D), jnp.float32),
        pltpu.VMEM_SHARED((N_PAD, D), jnp.float32),
        pltpu.SemaphoreType.DMA((NBUF_H,)),
        pltpu.SemaphoreType.DMA((NBUF_H,)),
        pltpu.SemaphoreType.DMA((NBUF_H,)),
        pltpu.SemaphoreType.DMA,
    ],
)


_R23_BLK = 2560


def _tc_r23_body(r1_ref, wr1_ref, wr2_ref, r2_ref, r3_ref):
    r2 = jnp.tanh(jnp.dot(r1_ref[...], wr1_ref[...], preferred_element_type=jnp.float32))
    r3 = jnp.tanh(jnp.dot(r2, wr2_ref[...], preferred_element_type=jnp.float32))
    r2_ref[...] = r2
    r3_ref[...] = r3


def _tc_r23(r1, wr1, wr2):
    return pl.pallas_call(
        _tc_r23_body,
        grid=(N_EDGES // _R23_BLK,),
        in_specs=[
            pl.BlockSpec((_R23_BLK, D), lambda i: (i, 0)),
            pl.BlockSpec((D, D), lambda i: (0, 0)),
            pl.BlockSpec((D, D), lambda i: (0, 0)),
        ],
        out_specs=[
            pl.BlockSpec((_R23_BLK, D), lambda i: (i, 0)),
            pl.BlockSpec((_R23_BLK, D), lambda i: (i, 0)),
        ],
        out_shape=[
            jax.ShapeDtypeStruct((N_EDGES, D), jnp.float32),
            jax.ShapeDtypeStruct((N_EDGES, D), jnp.float32),
        ],
    )(r1, wr1, wr2)


_X_BLK = 2048


def _tc_xupd_body(sr_ref, w_ref, x_ref):
    d = sr_ref[0] - sr_ref[1]
    x_ref[...] = jnp.tanh(jnp.dot(d, w_ref[...], preferred_element_type=jnp.float32))


def _tc_xupd(sr, w):
    return pl.pallas_call(
        _tc_xupd_body,
        grid=(N_PAD // _X_BLK,),
        in_specs=[
            pl.BlockSpec((2, _X_BLK, D), lambda i: (0, i, 0)),
            pl.BlockSpec((D, D), lambda i: (0, 0)),
        ],
        out_specs=pl.BlockSpec((_X_BLK, D), lambda i: (i, 0)),
        out_shape=jax.ShapeDtypeStruct((N_PAD, D), jnp.float32),
    )(sr, w)


_H_BLK = 2000


def _tc_head_body(sr2_ref, r3p_ref, w2_ref, m2_ref, fcw_ref, fcb_ref,
                  fc1w_ref, fc1b_ref, out_ref):
    d = sr2_ref[0] - sr2_ref[1]
    x2 = jnp.tanh(jnp.dot(d, w2_ref[...], preferred_element_type=jnp.float32))
    rn = r3p_ref[0] + r3p_ref[1]
    h = jnp.concatenate([x2, rn], axis=1)
    h = jnp.maximum(jnp.dot(h, m2_ref[...], preferred_element_type=jnp.float32), 0.0)
    h = lax.dot_general(h, fcw_ref[...], (((1,), (1,)), ((), ())),
                        preferred_element_type=jnp.float32) + fcb_ref[...]
    h = jnp.maximum(h, 0.0)
    out_ref[...] = lax.dot_general(h, fc1w_ref[...], (((1,), (1,)), ((), ())),
                                   preferred_element_type=jnp.float32) + fc1b_ref[...]


def _tc_head(sr2, r3p, w2, m2, fc_w, fc_b, fc1_w, fc1_b):
    return pl.pallas_call(
        _tc_head_body,
        grid=(N_NODES // _H_BLK,),
        in_specs=[
            pl.BlockSpec((2, _H_BLK, D), lambda i: (0, i, 0)),
            pl.BlockSpec((2, _H_BLK, D), lambda i: (0, i, 0)),
            pl.BlockSpec((D, D), lambda i: (0, 0)),
            pl.BlockSpec((2 * D, FLAT), lambda i: (0, 0)),
            pl.BlockSpec((D, FLAT), lambda i: (0, 0)),
            pl.BlockSpec((1, D), lambda i: (0, 0)),
            pl.BlockSpec((N_CLASSES, D), lambda i: (0, 0)),
            pl.BlockSpec((1, N_CLASSES), lambda i: (0, 0)),
        ],
        out_specs=pl.BlockSpec((_H_BLK, N_CLASSES), lambda i: (i, 0)),
        out_shape=jax.ShapeDtypeStruct((N_NODES, N_CLASSES), jnp.float32),
    )(sr2, r3p, w2, m2, fc_w, fc_b, fc1_w, fc1_b)


_A_CONV = np.zeros((16, 10, KER), np.float32)
for _i in range(10):
    for _a in range(KER):
        _A_CONV[_i + _a, _i, _a] = 1.0


def _build_m2(conv_w):
    t = jnp.einsum('fab,yia,xjb->yxfij', conv_w[:, 0], _A_CONV, _A_CONV,
                   preferred_element_type=jnp.float32)
    return t.reshape(16, 8, 2, 16, 10, 10).transpose(2, 0, 1, 3, 4, 5).reshape(2 * D, FLAT)


def kernel(node_features, edge_features, W1, Wr1, W2, Wr2, conv_w, fc_w, fc_b,
           fc1_w, fc1_b, edge_index):
    src = edge_index[0].astype(jnp.int32)
    dst = edge_index[1].astype(jnp.int32)
    zeros128 = jnp.zeros((128, D), jnp.float32)

    m2 = _build_m2(conv_w)
    r2, r3 = _tc_r23(edge_features, Wr1, Wr2)
    sr1 = _sc_dual(node_features, edge_features, src, dst, zeros128)
    x1 = _tc_xupd(sr1, W1)
    sr2 = _sc_dual(x1, r2, src, dst, zeros128)
    r3p = _sc_linear(r3, dst, zeros128)
    return _tc_head(sr2, r3p, W2, m2, fc_w, fc_b.reshape(1, D),
                    fc1_w, fc1_b.reshape(1, N_CLASSES))


# speedup vs baseline: 5.6043x; 1.1032x over previous
import jax
import jax.numpy as jnp
import numpy as np
from jax import lax
from jax.experimental import pallas as pl
from jax.experimental.pallas import tpu as pltpu
from jax.experimental.pallas import tpu_sc as plsc

N_NODES = 10000
N_EDGES = 320000
D = 128
N_CLASSES = 512
FLAT = 1600
KER = 7

N_PAD = 10240
ROWS_PER_TILE = N_PAD // 16
CHUNK = 128

EDGES_PER_TILE = N_EDGES // 16
FULL_CHUNKS = EDGES_PER_TILE // CHUNK
TAIL = EDGES_PER_TILE - FULL_CHUNKS * CHUNK
NBUF = 2

EDGES_PER_TILE_H = N_EDGES // 32
FULL_CHUNKS_H = EDGES_PER_TILE_H // CHUNK
TAIL_H = EDGES_PER_TILE_H - FULL_CHUNKS_H * CHUNK

_mesh = plsc.VectorSubcoreMesh(core_axis_name="c", subcore_axis_name="s")


def _zero_acc(zeros_hbm, acc_sh, sid, osem):
    for k in range(ROWS_PER_TILE // 128):
        pltpu.async_copy(zeros_hbm, acc_sh.at[pl.ds(sid * ROWS_PER_TILE + k * 128, 128)], osem)
    for k in range(ROWS_PER_TILE // 128):
        pltpu.make_async_copy(zeros_hbm, acc_sh.at[pl.ds(sid * ROWS_PER_TILE, 128)], osem).wait()


def _copy_out(acc_sh, out_hbm, cid, sid, osem):
    for k in range(ROWS_PER_TILE // 128):
        row = sid * ROWS_PER_TILE + k * 128
        pltpu.async_copy(acc_sh.at[pl.ds(row, 128)], out_hbm.at[cid, pl.ds(row, 128)], osem)
    for k in range(ROWS_PER_TILE // 128):
        pltpu.make_async_copy(acc_sh.at[pl.ds(0, 128)], out_hbm.at[cid, pl.ds(0, 128)], osem).wait()


NIDX = 4


def _gather_sweep(x_hbm, src_hbm, dst_hbm, acc_sh, src_v, dst_v, vals_v,
                  isem, gsem, ssem, base, full_chunks):
    last_off = base + (full_chunks - 1) * CHUNK
    for j in range(2):
        off = base + j * CHUNK
        pltpu.async_copy(src_hbm.at[pl.ds(off, CHUNK)], src_v.at[j], isem.at[j])
        pltpu.async_copy(dst_hbm.at[pl.ds(off, CHUNK)], dst_v.at[j], isem.at[j])

    def grp(g, carry):
        for b in range(NBUF):
            j = g * NBUF + b
            ib = (g * NBUF + b) % NIDX
            ib2 = (ib + 2) % NIDX

            @pl.when(g > 0)
            def _():
                pltpu.make_async_copy(vals_v.at[b], acc_sh.at[dst_v.at[ib]], ssem.at[b]).wait()
            offp = jnp.minimum(base + (j + 2) * CHUNK, last_off)
            pltpu.async_copy(src_hbm.at[pl.ds(offp, CHUNK)], src_v.at[ib2], isem.at[ib2])
            pltpu.async_copy(dst_hbm.at[pl.ds(offp, CHUNK)], dst_v.at[ib2], isem.at[ib2])
            pltpu.make_async_copy(src_hbm.at[pl.ds(0, CHUNK)], src_v.at[ib], isem.at[ib]).wait()
            pltpu.make_async_copy(dst_hbm.at[pl.ds(0, CHUNK)], dst_v.at[ib], isem.at[ib]).wait()
            pltpu.async_copy(x_hbm.at[src_v.at[ib]], vals_v.at[b], gsem.at[b])
        for b in range(NBUF):
            ib = (g * NBUF + b) % NIDX
            pltpu.make_async_copy(x_hbm.at[src_v.at[ib]], vals_v.at[b], gsem.at[b]).wait()
            pltpu.async_copy(vals_v.at[b], acc_sh.at[dst_v.at[ib]], ssem.at[b], add=True)
        return carry
    lax.fori_loop(0, full_chunks // NBUF, grp, 0)
    for b in range(NBUF):
        ib = (full_chunks - NBUF + b) % NIDX
        pltpu.make_async_copy(vals_v.at[b], acc_sh.at[dst_v.at[ib]], ssem.at[b]).wait()
        ibx = (full_chunks + b) % NIDX
        pltpu.make_async_copy(src_hbm.at[pl.ds(0, CHUNK)], src_v.at[ibx], isem.at[ibx]).wait()
        pltpu.make_async_copy(dst_hbm.at[pl.ds(0, CHUNK)], dst_v.at[ibx], isem.at[ibx]).wait()


def _linear_sweep(vals_hbm, dst_hbm, acc_sh, dst_v, vals_v,
                  isem, gsem, ssem, base, full_chunks):
    last_off = base + (full_chunks - 1) * CHUNK
    for j in range(2):
        pltpu.async_copy(dst_hbm.at[pl.ds(base + j * CHUNK, CHUNK)], dst_v.at[j], isem.at[j])

    def grp(g, carry):
        for b in range(NBUF):
            j = g * NBUF + b
            ib = j % NIDX
            ib2 = (ib + 2) % NIDX

            @pl.when(g > 0)
            def _():
                pltpu.make_async_copy(vals_v.at[b], acc_sh.at[dst_v.at[ib]], ssem.at[b]).wait()
            offp = jnp.minimum(base + (j + 2) * CHUNK, last_off)
            pltpu.async_copy(dst_hbm.at[pl.ds(offp, CHUNK)], dst_v.at[ib2], isem.at[ib2])
            off = base + j * CHUNK
            pltpu.async_copy(vals_hbm.at[pl.ds(off, CHUNK)], vals_v.at[b], gsem.at[b])
        for b in range(NBUF):
            ib = (g * NBUF + b) % NIDX
            pltpu.make_async_copy(vals_hbm.at[pl.ds(0, CHUNK)], vals_v.at[b], gsem.at[b]).wait()
            pltpu.make_async_copy(dst_hbm.at[pl.ds(0, CHUNK)], dst_v.at[ib], isem.at[ib]).wait()
            pltpu.async_copy(vals_v.at[b], acc_sh.at[dst_v.at[ib]], ssem.at[b], add=True)
        return carry
    lax.fori_loop(0, full_chunks // NBUF, grp, 0)
    for b in range(NBUF):
        ib = (full_chunks - NBUF + b) % NIDX
        pltpu.make_async_copy(vals_v.at[b], acc_sh.at[dst_v.at[ib]], ssem.at[b]).wait()
        ibx = (full_chunks + b) % NIDX
        pltpu.make_async_copy(dst_hbm.at[pl.ds(0, CHUNK)], dst_v.at[ibx], isem.at[ibx]).wait()


def _gather_tail(x_hbm, src_hbm, dst_hbm, acc_sh, srct_v, dstt_v, vals_v, gsem, off, tail):
    pltpu.sync_copy(src_hbm.at[pl.ds(off, tail)], srct_v)
    pltpu.async_copy(x_hbm.at[srct_v], vals_v.at[0, pl.ds(0, tail)], gsem.at[0]).wait()
    pltpu.sync_copy(dst_hbm.at[pl.ds(off, tail)], dstt_v)
    pltpu.sync_copy(vals_v.at[0, pl.ds(0, tail)], acc_sh.at[dstt_v], add=True)


def _linear_tail(vals_hbm, dst_hbm, acc_sh, dstt_v, vals_v, off, tail):
    pltpu.sync_copy(vals_hbm.at[pl.ds(off, tail)], vals_v.at[0, pl.ds(0, tail)])
    pltpu.sync_copy(dst_hbm.at[pl.ds(off, tail)], dstt_v)
    pltpu.sync_copy(vals_v.at[0, pl.ds(0, tail)], acc_sh.at[dstt_v], add=True)


def _sc_dual_body(x_hbm, vals_hbm, src_hbm, dst_hbm, zeros_hbm, out_hbm,
                  src_v, dst_v, srct_v, dstt_v, vals_v, acc_sh,
                  isem, gsem, ssem, osem):
    cid = lax.axis_index("c")
    sid = lax.axis_index("s")
    _zero_acc(zeros_hbm, acc_sh, sid, osem)
    plsc.subcore_barrier()
    base = sid * EDGES_PER_TILE
    toff = base + FULL_CHUNKS * CHUNK

    @pl.when(cid == 0)
    def _():
        _gather_sweep(x_hbm, src_hbm, dst_hbm, acc_sh, src_v, dst_v, vals_v,
                      isem, gsem, ssem, base, FULL_CHUNKS)
        _gather_tail(x_hbm, src_hbm, dst_hbm, acc_sh, srct_v, dstt_v, vals_v, gsem, toff, TAIL)

    @pl.when(cid == 1)
    def _():
        _linear_sweep(vals_hbm, dst_hbm, acc_sh, dst_v, vals_v,
                      isem, gsem, ssem, base, FULL_CHUNKS)
        _linear_tail(vals_hbm, dst_hbm, acc_sh, dstt_v, vals_v, toff, TAIL)

    plsc.subcore_barrier()
    _copy_out(acc_sh, out_hbm, cid, sid, osem)


_sc_dual = pl.kernel(
    _sc_dual_body,
    out_type=jax.ShapeDtypeStruct((2, N_PAD, D), jnp.float32),
    mesh=_mesh,
    scratch_types=[
        pltpu.VMEM((NIDX, CHUNK), jnp.int32),
        pltpu.VMEM((NIDX, CHUNK), jnp.int32),
        pltpu.VMEM((TAIL,), jnp.int32),
        pltpu.VMEM((TAIL,), jnp.int32),
        pltpu.VMEM((NBUF, CHUNK, D), jnp.float32),
        pltpu.VMEM_SHARED((N_PAD, D), jnp.float32),
        pltpu.SemaphoreType.DMA((NIDX,)),
        pltpu.SemaphoreType.DMA((NBUF,)),
        pltpu.SemaphoreType.DMA((NBUF,)),
        pltpu.SemaphoreType.DMA,
    ],
)


def _sc_linear_body(vals_hbm, dst_hbm, zeros_hbm, out_hbm,
                    dst_v, dstt_v, vals_v, acc_sh, isem, gsem, ssem, osem):
    cid = lax.axis_index("c")
    sid = lax.axis_index("s")
    _zero_acc(zeros_hbm, acc_sh, sid, osem)
    plsc.subcore_barrier()
    base = cid * (N_EDGES // 2) + sid * EDGES_PER_TILE_H
    _linear_sweep(vals_hbm, dst_hbm, acc_sh, dst_v, vals_v,
                  isem, gsem, ssem, base, FULL_CHUNKS_H)
    _linear_tail(vals_hbm, dst_hbm, acc_sh, dstt_v, vals_v,
                 base + FULL_CHUNKS_H * CHUNK, TAIL_H)
    plsc.subcore_barrier()
    _copy_out(acc_sh, out_hbm, cid, sid, osem)


_sc_linear = pl.kernel(
    _sc_linear_body,
    out_type=jax.ShapeDtypeStruct((2, N_PAD, D), jnp.float32),
    mesh=_mesh,
    scratch_types=[
        pltpu.VMEM((NIDX, CHUNK), jnp.int32),
        pltpu.VMEM((TAIL_H,), jnp.int32),
        pltpu.VMEM((NBUF, CHUNK, D), jnp.float32),
        pltpu.VMEM_SHARED((N_PAD, D), jnp.float32),
        pltpu.SemaphoreType.DMA((NIDX,)),
        pltpu.SemaphoreType.DMA((NBUF,)),
        pltpu.SemaphoreType.DMA((NBUF,)),
        pltpu.SemaphoreType.DMA,
    ],
)


_R23_BLK = 2560


def _tc_r23_body(r1_ref, wr1_ref, wr2_ref, r2_ref, r3_ref):
    r2 = jnp.tanh(jnp.dot(r1_ref[...], wr1_ref[...], preferred_element_type=jnp.float32))
    r3 = jnp.tanh(jnp.dot(r2, wr2_ref[...], preferred_element_type=jnp.float32))
    r2_ref[...] = r2
    r3_ref[...] = r3


def _tc_r23(r1, wr1, wr2):
    return pl.pallas_call(
        _tc_r23_body,
        grid=(N_EDGES // _R23_BLK,),
        in_specs=[
            pl.BlockSpec((_R23_BLK, D), lambda i: (i, 0)),
            pl.BlockSpec((D, D), lambda i: (0, 0)),
            pl.BlockSpec((D, D), lambda i: (0, 0)),
        ],
        out_specs=[
            pl.BlockSpec((_R23_BLK, D), lambda i: (i, 0)),
            pl.BlockSpec((_R23_BLK, D), lambda i: (i, 0)),
        ],
        out_shape=[
            jax.ShapeDtypeStruct((N_EDGES, D), jnp.float32),
            jax.ShapeDtypeStruct((N_EDGES, D), jnp.float32),
        ],
    )(r1, wr1, wr2)


_X_BLK = 2048


def _tc_xupd_body(sr_ref, w_ref, x_ref):
    d = sr_ref[0] - sr_ref[1]
    x_ref[...] = jnp.tanh(jnp.dot(d, w_ref[...], preferred_element_type=jnp.float32))


def _tc_xupd(sr, w):
    return pl.pallas_call(
        _tc_xupd_body,
        grid=(N_PAD // _X_BLK,),
        in_specs=[
            pl.BlockSpec((2, _X_BLK, D), lambda i: (0, i, 0)),
            pl.BlockSpec((D, D), lambda i: (0, 0)),
        ],
        out_specs=pl.BlockSpec((_X_BLK, D), lambda i: (i, 0)),
        out_shape=jax.ShapeDtypeStruct((N_PAD, D), jnp.float32),
    )(sr, w)


_H_BLK = 2000


def _tc_head_body(sr2_ref, r3p_ref, w2_ref, m2_ref, fcw_ref, fcb_ref,
                  fc1w_ref, fc1b_ref, out_ref):
    d = sr2_ref[0] - sr2_ref[1]
    x2 = jnp.tanh(jnp.dot(d, w2_ref[...], preferred_element_type=jnp.float32))
    rn = r3p_ref[0] + r3p_ref[1]
    h = jnp.concatenate([x2, rn], axis=1)
    h = jnp.maximum(jnp.dot(h, m2_ref[...], preferred_element_type=jnp.float32), 0.0)
    h = lax.dot_general(h, fcw_ref[...], (((1,), (1,)), ((), ())),
                        preferred_element_type=jnp.float32) + fcb_ref[...]
    h = jnp.maximum(h, 0.0)
    out_ref[...] = lax.dot_general(h, fc1w_ref[...], (((1,), (1,)), ((), ())),
                                   preferred_element_type=jnp.float32) + fc1b_ref[...]


def _tc_head(sr2, r3p, w2, m2, fc_w, fc_b, fc1_w, fc1_b):
    return pl.pallas_call(
        _tc_head_body,
        grid=(N_NODES // _H_BLK,),
        in_specs=[
            pl.BlockSpec((2, _H_BLK, D), lambda i: (0, i, 0)),
            pl.BlockSpec((2, _H_BLK, D), lambda i: (0, i, 0)),
            pl.BlockSpec((D, D), lambda i: (0, 0)),
            pl.BlockSpec((2 * D, FLAT), lambda i: (0, 0)),
            pl.BlockSpec((D, FLAT), lambda i: (0, 0)),
            pl.BlockSpec((1, D), lambda i: (0, 0)),
            pl.BlockSpec((N_CLASSES, D), lambda i: (0, 0)),
            pl.BlockSpec((1, N_CLASSES), lambda i: (0, 0)),
        ],
        out_specs=pl.BlockSpec((_H_BLK, N_CLASSES), lambda i: (i, 0)),
        out_shape=jax.ShapeDtypeStruct((N_NODES, N_CLASSES), jnp.float32),
    )(sr2, r3p, w2, m2, fc_w, fc_b, fc1_w, fc1_b)


_A_CONV = np.zeros((16, 10, KER), np.float32)
for _i in range(10):
    for _a in range(KER):
        _A_CONV[_i + _a, _i, _a] = 1.0


def _build_m2(conv_w):
    t = jnp.einsum('fab,yia,xjb->yxfij', conv_w[:, 0], _A_CONV, _A_CONV,
                   preferred_element_type=jnp.float32)
    return t.reshape(16, 8, 2, 16, 10, 10).transpose(2, 0, 1, 3, 4, 5).reshape(2 * D, FLAT)


def kernel(node_features, edge_features, W1, Wr1, W2, Wr2, conv_w, fc_w, fc_b,
           fc1_w, fc1_b, edge_index):
    src = edge_index[0].astype(jnp.int32)
    dst = edge_index[1].astype(jnp.int32)
    zeros128 = jnp.zeros((128, D), jnp.float32)

    m2 = _build_m2(conv_w)
    r2, r3 = _tc_r23(edge_features, Wr1, Wr2)
    sr1 = _sc_dual(node_features, edge_features, src, dst, zeros128)
    x1 = _tc_xupd(sr1, W1)
    sr2 = _sc_dual(x1, r2, src, dst, zeros128)
    r3p = _sc_linear(r3, dst, zeros128)
    return _tc_head(sr2, r3p, W2, m2, fc_w, fc_b.reshape(1, D),
                    fc1_w, fc1_b.reshape(1, N_CLASSES))
